# Initial kernel scaffold; baseline (speedup 1.0000x reference)
import jax
import jax.numpy as jnp
from jax.experimental import pallas as pl


def kernel(x, edge_index, edge_attr, batch, c1_Wf, c1_bf, c1_Ws, c1_bs, proj_W, proj_b, c2_Wf, c2_bf, c2_Ws, c2_bs, c3_Wf, c3_bf, c3_Ws, c3_bs, fc1_W, fc1_b, ln_g, ln_b, head_W, head_b):
    raise NotImplementedError("write your pallas kernel here")


# speedup vs baseline: 1.6510x; 1.6510x over previous
import functools

import jax
import jax.numpy as jnp
from jax import lax
from jax.experimental import pallas as pl
from jax.experimental.pallas import tpu as pltpu
from jax.experimental.pallas import tpu_sc as plsc

N = 50000
E = 800000
NODE_DIM = 3
EDGE_DIM = 16
HID = 64
NUM_GRAPHS = 512

NP = 51200
EP = 802816
NR = EP // 128
PAD_NODE = N

NW = 32
RW = NR // NW
BE = 2048
EG = EP // BE
BN = 2048
NG = NP // BN

_mesh = plsc.VectorSubcoreMesh(core_axis_name="c", subcore_axis_name="s")
_f32 = jnp.float32


@functools.partial(
    pl.kernel,
    out_type=jax.ShapeDtypeStruct((EP, 16), _f32),
    mesh=_mesh,
    compiler_params=pltpu.CompilerParams(use_tc_tiling_on_sc=False),
    scratch_types=[
        pltpu.VMEM((4, 128), jnp.int32),
        pltpu.VMEM((4, 128), jnp.int32),
        pltpu.VMEM((512, 16), _f32),
        pltpu.SemaphoreType.DMA,
    ],
)
def _sc_gather1(d1, s1, dst2, src2, out, idxd, idxs, buf, sem):
    c = lax.axis_index("c")
    s = lax.axis_index("s")
    w = s * 2 + c

    def body(i, _):
        rbase = w * RW + i * 4
        pltpu.sync_copy(dst2.at[pl.ds(rbase, 4), :], idxd)
        pltpu.sync_copy(src2.at[pl.ds(rbase, 4), :], idxs)
        cps = [pltpu.async_copy(d1.at[idxd.at[j]],
                                buf.at[pl.ds(j * 128, 128), :], sem)
               for j in range(4)]
        for cp in cps:
            cp.wait()
        cps = [pltpu.async_copy(s1.at[idxs.at[j]],
                                buf.at[pl.ds(j * 128, 128), :], sem, add=True)
               for j in range(4)]
        for cp in cps:
            cp.wait()
        pltpu.sync_copy(buf, out.at[pl.ds(rbase * 128, 512), :])
        return 0

    lax.fori_loop(0, RW // 4, body, 0)


_G23_OUT = [jax.ShapeDtypeStruct((EP, 32), _f32) for _ in range(4)]


@functools.partial(
    pl.kernel,
    out_type=_G23_OUT,
    mesh=_mesh,
    compiler_params=pltpu.CompilerParams(use_tc_tiling_on_sc=False),
    scratch_types=[
        pltpu.VMEM((4, 128), jnp.int32),
        pltpu.VMEM((4, 128), jnp.int32),
        pltpu.VMEM((512, 32), _f32),
        pltpu.VMEM((512, 32), _f32),
        pltpu.VMEM((512, 32), _f32),
        pltpu.VMEM((512, 32), _f32),
        pltpu.SemaphoreType.DMA,
    ],
)
def _sc_gather23(h0, h1, dst2, src2, od0, od1, os0, os1,
                 idxd, idxs, b0, b1, b2, b3, sem):
    c = lax.axis_index("c")
    s = lax.axis_index("s")
    w = s * 2 + c

    def body(i, _):
        rbase = w * RW + i * 4
        pltpu.sync_copy(dst2.at[pl.ds(rbase, 4), :], idxd)
        pltpu.sync_copy(src2.at[pl.ds(rbase, 4), :], idxs)
        cps = []
        for j in range(4):
            sl = pl.ds(j * 128, 128)
            cps.append(pltpu.async_copy(h0.at[idxd.at[j]], b0.at[sl, :], sem))
            cps.append(pltpu.async_copy(h1.at[idxd.at[j]], b1.at[sl, :], sem))
            cps.append(pltpu.async_copy(h0.at[idxs.at[j]], b2.at[sl, :], sem))
            cps.append(pltpu.async_copy(h1.at[idxs.at[j]], b3.at[sl, :], sem))
        for cp in cps:
            cp.wait()
        ebase = rbase * 128
        pltpu.sync_copy(b0, od0.at[pl.ds(ebase, 512), :])
        pltpu.sync_copy(b1, od1.at[pl.ds(ebase, 512), :])
        pltpu.sync_copy(b2, os0.at[pl.ds(ebase, 512), :])
        pltpu.sync_copy(b3, os1.at[pl.ds(ebase, 512), :])
        return 0

    lax.fori_loop(0, RW // 4, body, 0)


@functools.partial(
    pl.kernel,
    out_type=[jax.ShapeDtypeStruct((NP, 8), _f32),
              jax.ShapeDtypeStruct((NP, 8), _f32)],
    mesh=_mesh,
    compiler_params=pltpu.CompilerParams(use_tc_tiling_on_sc=False),
    scratch_types=[
        pltpu.VMEM((4, 128), jnp.int32),
        pltpu.VMEM((512, 8), _f32),
        pltpu.VMEM_SHARED((NP, 8), _f32),
    ],
)
def _sc_scatter1(m1, dst2, x8, zn8, agg0, agg1, idx, mbuf, acc):
    c = lax.axis_index("c")
    s = lax.axis_index("s")
    w = s * 2 + c
    rows = pl.ds(s * 3200, 3200)

    @pl.when(c == 0)
    def _():
        pltpu.sync_copy(x8.at[rows, :], acc.at[rows, :])

    @pl.when(c == 1)
    def _():
        pltpu.sync_copy(zn8.at[rows, :], acc.at[rows, :])

    plsc.subcore_barrier()

    def body(i, _):
        rbase = w * RW + i * 4
        pltpu.sync_copy(dst2.at[pl.ds(rbase, 4), :], idx)
        pltpu.sync_copy(m1.at[pl.ds(rbase * 128, 512), :], mbuf)
        for j in range(4):
            pltpu.sync_copy(mbuf.at[pl.ds(j * 128, 128), :],
                            acc.at[idx.at[j]], add=True)
        return 0

    lax.fori_loop(0, RW // 4, body, 0)
    plsc.subcore_barrier()

    @pl.when(c == 0)
    def _():
        pltpu.sync_copy(acc.at[rows, :], agg0.at[rows, :])

    @pl.when(c == 1)
    def _():
        pltpu.sync_copy(acc.at[rows, :], agg1.at[rows, :])


@functools.partial(
    pl.kernel,
    out_type=[jax.ShapeDtypeStruct((NP, 32), _f32),
              jax.ShapeDtypeStruct((NP, 32), _f32)],
    mesh=_mesh,
    compiler_params=pltpu.CompilerParams(use_tc_tiling_on_sc=False),
    scratch_types=[
        pltpu.VMEM((8, 128), jnp.int32),
        pltpu.VMEM((1024, 16), _f32),
        pltpu.VMEM_SHARED((NP, 16), _f32),
    ],
)
def _sc_scatter23(m0, m1, dst2, h0, h1, acc0, acc1, idx, mbuf, acc):
    c = lax.axis_index("c")
    s = lax.axis_index("s")
    rows = pl.ds(s * 3200, 3200)
    rpt = NR // 16

    for q in range(2):
        cols = pl.ds(q * 16, 16)

        @pl.when(c == 0)
        def _():
            pltpu.sync_copy(h0.at[rows, cols], acc.at[rows, :])

        @pl.when(c == 1)
        def _():
            pltpu.sync_copy(h1.at[rows, cols], acc.at[rows, :])

        plsc.subcore_barrier()

        def body(i, _):
            rbase = s * rpt + i * 8
            pltpu.sync_copy(dst2.at[pl.ds(rbase, 8), :], idx)

            @pl.when(c == 0)
            def _():
                pltpu.sync_copy(m0.at[pl.ds(rbase * 128, 1024), cols], mbuf)

            @pl.when(c == 1)
            def _():
                pltpu.sync_copy(m1.at[pl.ds(rbase * 128, 1024), cols], mbuf)

            for j in range(8):
                pltpu.sync_copy(mbuf.at[pl.ds(j * 128, 128), :],
                                acc.at[idx.at[j]], add=True)
            return 0

        lax.fori_loop(0, rpt // 8, body, 0)
        plsc.subcore_barrier()

        @pl.when(c == 0)
        def _():
            pltpu.sync_copy(acc.at[rows, :], acc0.at[rows, cols])

        @pl.when(c == 1)
        def _():
            pltpu.sync_copy(acc.at[rows, :], acc1.at[rows, cols])

        plsc.subcore_barrier()


_PREC = jax.lax.Precision.HIGHEST


def _dot(a, b):
    return jnp.dot(a, b, preferred_element_type=_f32, precision=_PREC)


def _sigmoid(x):
    return 1.0 / (1.0 + jnp.exp(-x))


def _softplus(x):
    return jnp.maximum(x, 0.0) + jnp.log1p(jnp.exp(-jnp.abs(x)))


def _nodeprep1_body(x8, w1d, w1s, d1, s1):
    xv = x8[...]
    d1[...] = _dot(xv, w1d[...])
    s1[...] = _dot(xv, w1s[...])


def _edge1_body(asum, ea, we1, b1, m1):
    a = asum[...] + _dot(ea[...], we1[...]) + b1[...]
    m1[...] = _sigmoid(a[:, 0:8]) * _softplus(a[:, 8:16])


def _proj_body(agg0, agg1, pw, pb, g, b, h0, h1):
    xs = agg0[...] + agg1[...]
    h = _dot(xs, pw[...]) + pb[...]
    mu = jnp.mean(h, axis=1, keepdims=True)
    var = jnp.mean((h - mu) ** 2, axis=1, keepdims=True)
    h = (h - mu) / jnp.sqrt(var + 1e-5) * g[...] + b[...]
    h = jnp.maximum(h, 0.0)
    h0[...] = h[:, 0:32]
    h1[...] = h[:, 32:64]


def _edge23_body(gd0, gd1, gs0, gs1, ea, wd, ws, we, bb, m0, m1):
    wdv = wd[...]
    wsv = ws[...]
    a = (_dot(gd0[...], wdv[0:32, :]) + _dot(gd1[...], wdv[32:64, :])
         + _dot(gs0[...], wsv[0:32, :]) + _dot(gs1[...], wsv[32:64, :])
         + _dot(ea[...], we[...]) + bb[...])
    m = _sigmoid(a[:, 0:64]) * _softplus(a[:, 64:128])
    m0[...] = m[:, 0:32]
    m1[...] = m[:, 32:64]


def _act_body(a0, a1, h0, h1):
    h0[...] = jnp.maximum(jnp.clip(a0[...], -1e6, 1e6), 0.0)
    h1[...] = jnp.maximum(jnp.clip(a1[...], -1e6, 1e6), 0.0)


def _poolact_body(a0, a1, bt, out):
    i = pl.program_id(0)
    h = jnp.concatenate([a0[...], a1[...]], axis=1)
    h = jnp.maximum(jnp.clip(h, -1e6, 1e6), 0.0)
    haug = jnp.concatenate([h, jnp.ones((h.shape[0], 8), _f32)], axis=1)
    gids = bt[...].reshape(h.shape[0], 1)
    iota = jax.lax.broadcasted_iota(jnp.int32, (h.shape[0], NUM_GRAPHS), 1)
    onehot = (gids == iota).astype(_f32)
    contrib = jax.lax.dot_general(
        onehot, haug, (((0,), (0,)), ((), ())),
        preferred_element_type=_f32, precision=_PREC)

    @pl.when(i == 0)
    def _():
        out[...] = jnp.zeros_like(out)

    out[...] += contrib


def _head_body(pa, fw, fb, g, b, hw, hb, out):
    p = pa[...]
    pooled = p[:, 0:64] / jnp.maximum(p[:, 64:65], 1.0)
    gg = _dot(pooled, fw[...]) + fb[...]
    mu = jnp.mean(gg, axis=1, keepdims=True)
    var = jnp.mean((gg - mu) ** 2, axis=1, keepdims=True)
    gg = (gg - mu) / jnp.sqrt(var + 1e-5) * g[...] + b[...]
    gg = jnp.clip(jnp.maximum(gg, 0.0), -1e6, 1e6)
    out[...] = _dot(gg, hw[...]) + hb[...]


def _full(shape):
    return pl.BlockSpec(shape, lambda i: (0, 0))


def _nodeprep1(x8p, w1d, w1s):
    return pl.pallas_call(
        _nodeprep1_body,
        grid=(NG,),
        in_specs=[pl.BlockSpec((BN, 8), lambda i: (i, 0)),
                  _full((8, 16)), _full((8, 16))],
        out_specs=[pl.BlockSpec((BN, 16), lambda i: (i, 0))] * 2,
        out_shape=[jax.ShapeDtypeStruct((NP, 16), _f32)] * 2,
    )(x8p, w1d, w1s)


def _edge1(asum, eap, we1, b1):
    return pl.pallas_call(
        _edge1_body,
        grid=(EG,),
        in_specs=[pl.BlockSpec((BE, 16), lambda i: (i, 0)),
                  pl.BlockSpec((BE, 16), lambda i: (i, 0)),
                  _full((16, 16)), _full((1, 16))],
        out_specs=pl.BlockSpec((BE, 8), lambda i: (i, 0)),
        out_shape=jax.ShapeDtypeStruct((EP, 8), _f32),
    )(asum, eap, we1, b1)


def _proj(agg0, agg1, pw, pb, g, b):
    return pl.pallas_call(
        _proj_body,
        grid=(NG,),
        in_specs=[pl.BlockSpec((BN, 8), lambda i: (i, 0)),
                  pl.BlockSpec((BN, 8), lambda i: (i, 0)),
                  _full((8, 64)), _full((1, 64)), _full((1, 64)),
                  _full((1, 64))],
        out_specs=[pl.BlockSpec((BN, 32), lambda i: (i, 0))] * 2,
        out_shape=[jax.ShapeDtypeStruct((NP, 32), _f32)] * 2,
    )(agg0, agg1, pw, pb, g, b)


def _edge23(gd0, gd1, gs0, gs1, eap, wd, ws, we, bb):
    espec = pl.BlockSpec((BE, 32), lambda i: (i, 0))
    return pl.pallas_call(
        _edge23_body,
        grid=(EG,),
        in_specs=[espec, espec, espec, espec,
                  pl.BlockSpec((BE, 16), lambda i: (i, 0)),
                  _full((64, 128)), _full((64, 128)), _full((16, 128)),
                  _full((1, 128))],
        out_specs=[espec] * 2,
        out_shape=[jax.ShapeDtypeStruct((EP, 32), _f32)] * 2,
    )(gd0, gd1, gs0, gs1, eap, wd, ws, we, bb)


def _act(a0, a1):
    nspec = pl.BlockSpec((BN, 32), lambda i: (i, 0))
    return pl.pallas_call(
        _act_body,
        grid=(NG,),
        in_specs=[nspec, nspec],
        out_specs=[nspec] * 2,
        out_shape=[jax.ShapeDtypeStruct((NP, 32), _f32)] * 2,
    )(a0, a1)


def _poolact(a0, a1, bt):
    nspec = pl.BlockSpec((2000, 32), lambda i: (i, 0))
    return pl.pallas_call(
        _poolact_body,
        grid=(25,),
        in_specs=[nspec, nspec,
                  pl.BlockSpec((1, 2000, 1), lambda i: (i, 0, 0))],
        out_specs=pl.BlockSpec((NUM_GRAPHS, 72), lambda i: (0, 0)),
        out_shape=jax.ShapeDtypeStruct((NUM_GRAPHS, 72), _f32),
    )(a0, a1, bt)


def _head(pa, fw, fb, g, b, hw, hb):
    return pl.pallas_call(
        _head_body,
        in_specs=[pl.BlockSpec((NUM_GRAPHS, 72), lambda: (0, 0)),
                  pl.BlockSpec((64, 64), lambda: (0, 0)),
                  pl.BlockSpec((1, 64), lambda: (0, 0)),
                  pl.BlockSpec((1, 64), lambda: (0, 0)),
                  pl.BlockSpec((1, 64), lambda: (0, 0)),
                  pl.BlockSpec((64, 8), lambda: (0, 0)),
                  pl.BlockSpec((1, 8), lambda: (0, 0))],
        out_specs=pl.BlockSpec((NUM_GRAPHS, 8), lambda: (0, 0)),
        out_shape=jax.ShapeDtypeStruct((NUM_GRAPHS, 8), _f32),
    )(pa, fw, fb, g, b, hw, hb)


def kernel(x, edge_index, edge_attr, batch, c1_Wf, c1_bf, c1_Ws, c1_bs,
           proj_W, proj_b, c2_Wf, c2_bf, c2_Ws, c2_bs, c3_Wf, c3_bf,
           c3_Ws, c3_bs, fc1_W, fc1_b, ln_g, ln_b, head_W, head_b):
    f32 = _f32
    x8p = jnp.zeros((NP, 8), f32).at[:N, :3].set(x)
    zn8 = jnp.zeros((NP, 8), f32)
    dst = edge_index[1].astype(jnp.int32)
    src = edge_index[0].astype(jnp.int32)
    padi = jnp.full((EP - E,), PAD_NODE, jnp.int32)
    dst2 = jnp.concatenate([dst, padi]).reshape(NR, 128)
    src2 = jnp.concatenate([src, padi]).reshape(NR, 128)
    eap = jnp.zeros((EP, 16), f32).at[:E, :].set(edge_attr)

    w1d = jnp.zeros((8, 16), f32).at[0:3, 0:3].set(c1_Wf[0:3]) \
                                 .at[0:3, 8:11].set(c1_Ws[0:3])
    w1s = jnp.zeros((8, 16), f32).at[0:3, 0:3].set(c1_Wf[3:6]) \
                                 .at[0:3, 8:11].set(c1_Ws[3:6])
    we1 = jnp.zeros((16, 16), f32).at[:, 0:3].set(c1_Wf[6:22]) \
                                  .at[:, 8:11].set(c1_Ws[6:22])
    b1 = jnp.zeros((1, 16), f32).at[0, 0:3].set(c1_bf).at[0, 8:11].set(c1_bs)

    pw8 = jnp.zeros((8, 64), f32).at[0:3, :].set(proj_W)
    pb = proj_b.reshape(1, 64)
    lg = ln_g.reshape(1, 64)
    lb = ln_b.reshape(1, 64)

    def wsplit(wf, wsm, bf, bs):
        wd = jnp.concatenate([wf[0:64], wsm[0:64]], axis=1)
        wsr = jnp.concatenate([wf[64:128], wsm[64:128]], axis=1)
        we = jnp.concatenate([wf[128:144], wsm[128:144]], axis=1)
        bb = jnp.concatenate([bf, bs]).reshape(1, 128)
        return wd, wsr, we, bb

    wd2, wsr2, we2, bb2 = wsplit(c2_Wf, c2_Ws, c2_bf, c2_bs)
    wd3, wsr3, we3, bb3 = wsplit(c3_Wf, c3_Ws, c3_bf, c3_bs)

    bt = batch.astype(jnp.int32).reshape(25, 2000, 1)
    fw = fc1_W
    fb = fc1_b.reshape(1, 64)
    hw8 = jnp.zeros((64, 8), f32).at[:, 0:5].set(head_W)
    hb8 = jnp.zeros((1, 8), f32).at[0, 0:5].set(head_b)

    d1, s1 = _nodeprep1(x8p, w1d, w1s)
    asum1 = _sc_gather1(d1, s1, dst2, src2)
    m1 = _edge1(asum1, eap, we1, b1)
    agg0, agg1 = _sc_scatter1(m1, dst2, x8p, zn8)
    h0, h1 = _proj(agg0, agg1, pw8, pb, lg, lb)

    gd0, gd1, gs0, gs1 = _sc_gather23(h0, h1, dst2, src2)
    m20, m21 = _edge23(gd0, gd1, gs0, gs1, eap, wd2, wsr2, we2, bb2)
    a20, a21 = _sc_scatter23(m20, m21, dst2, h0, h1)
    h20, h21 = _act(a20, a21)

    gd0, gd1, gs0, gs1 = _sc_gather23(h20, h21, dst2, src2)
    m30, m31 = _edge23(gd0, gd1, gs0, gs1, eap, wd3, wsr3, we3, bb3)
    a30, a31 = _sc_scatter23(m30, m31, dst2, h20, h21)

    pa = _poolact(a30, a31, bt)
    out8 = _head(pa, fw, fb, lg, lb, hw8, hb8)
    return out8[:, 0:5]


# speedup vs baseline: 3.4532x; 2.0916x over previous
import functools

import jax
import jax.numpy as jnp
from jax import lax
from jax.experimental import pallas as pl
from jax.experimental.pallas import tpu as pltpu
from jax.experimental.pallas import tpu_sc as plsc

N = 50000
E = 800000
HID = 64
NUM_GRAPHS = 512

NP = 51200
EP = 802816
NR = EP // 128
PAD_NODE = N

NW = 32
RW = NR // NW
BE = 2048
EG = EP // BE
BN = 2048
NG = NP // BN

_mesh = plsc.VectorSubcoreMesh(core_axis_name="c", subcore_axis_name="s")
_f32 = jnp.float32
_SCPARAMS = pltpu.CompilerParams(use_tc_tiling_on_sc=False)


@functools.partial(
    pl.kernel,
    out_type=jax.ShapeDtypeStruct((EP, 16), _f32),
    mesh=_mesh,
    compiler_params=_SCPARAMS,
    scratch_types=[
        pltpu.VMEM((4, 128), jnp.int32),
        pltpu.VMEM((4, 128), jnp.int32),
        pltpu.VMEM((512, 16), _f32),
        pltpu.SemaphoreType.DMA,
    ],
)
def _sc_gather1(d1, s1, dst2, src2, out, idxd, idxs, buf, sem):
    c = lax.axis_index("c")
    s = lax.axis_index("s")
    w = s * 2 + c

    def body(i, _):
        rbase = w * RW + i * 4
        pltpu.sync_copy(dst2.at[pl.ds(rbase, 4), :], idxd)
        pltpu.sync_copy(src2.at[pl.ds(rbase, 4), :], idxs)
        cps = [pltpu.async_copy(d1.at[idxd.at[j]],
                                buf.at[pl.ds(j * 128, 128), :], sem)
               for j in range(4)]
        for cp in cps:
            cp.wait()
        cps = [pltpu.async_copy(s1.at[idxs.at[j]],
                                buf.at[pl.ds(j * 128, 128), :], sem, add=True)
               for j in range(4)]
        for cp in cps:
            cp.wait()
        pltpu.sync_copy(buf, out.at[pl.ds(rbase * 128, 512), :])
        return 0

    lax.fori_loop(0, RW // 4, body, 0)


_KR = 2
_NCH = RW // _KR


@functools.partial(
    pl.kernel,
    out_type=jax.ShapeDtypeStruct((EP, 128), _f32),
    mesh=_mesh,
    compiler_params=_SCPARAMS,
    scratch_types=[
        pltpu.VMEM((_KR, 128), jnp.int32),
        pltpu.VMEM((_KR, 128), jnp.int32),
        pltpu.VMEM((_KR, 128), jnp.int32),
        pltpu.VMEM((_KR, 128), jnp.int32),
        pltpu.VMEM((_KR * 128, 64), _f32),
        pltpu.VMEM((_KR * 128, 64), _f32),
        pltpu.VMEM((_KR * 128, 64), _f32),
        pltpu.VMEM((_KR * 128, 64), _f32),
        pltpu.SemaphoreType.DMA,
        pltpu.SemaphoreType.DMA,
    ],
)
def _sc_gather23(h, dst2, src2, out,
                 ia0, is0, ia1, is1, bd0, bs0, bd1, bs1, sem0, sem1):
    c = lax.axis_index("c")
    s = lax.axis_index("s")
    w = s * 2 + c
    wbase = w * RW

    def load_and_fire(chunk, ia, is_, bd, bs, sem):
        rbase = wbase + chunk * _KR
        pltpu.sync_copy(dst2.at[pl.ds(rbase, _KR), :], ia)
        pltpu.sync_copy(src2.at[pl.ds(rbase, _KR), :], is_)
        for j in range(_KR):
            sl = pl.ds(j * 128, 128)
            pltpu.async_copy(h.at[ia.at[j]], bd.at[sl, :], sem)
            pltpu.async_copy(h.at[is_.at[j]], bs.at[sl, :], sem)

    def drain_and_write(chunk, ia, is_, bd, bs, sem):
        for j in range(_KR):
            sl = pl.ds(j * 128, 128)
            pltpu.make_async_copy(h.at[ia.at[j]], bd.at[sl, :], sem).wait()
            pltpu.make_async_copy(h.at[is_.at[j]], bs.at[sl, :], sem).wait()
        ebase = (wbase + chunk * _KR) * 128
        pltpu.sync_copy(bd, out.at[pl.ds(ebase, _KR * 128), pl.ds(0, 64)])
        pltpu.sync_copy(bs, out.at[pl.ds(ebase, _KR * 128), pl.ds(64, 64)])

    load_and_fire(0, ia0, is0, bd0, bs0, sem0)

    def body(g, _):
        c0 = 2 * g
        load_and_fire(c0 + 1, ia1, is1, bd1, bs1, sem1)
        drain_and_write(c0, ia0, is0, bd0, bs0, sem0)

        @pl.when(c0 + 2 < _NCH)
        def _():
            load_and_fire(c0 + 2, ia0, is0, bd0, bs0, sem0)

        drain_and_write(c0 + 1, ia1, is1, bd1, bs1, sem1)
        return 0

    lax.fori_loop(0, _NCH // 2, body, 0)


@functools.partial(
    pl.kernel,
    out_type=[jax.ShapeDtypeStruct((NP, 8), _f32),
              jax.ShapeDtypeStruct((NP, 8), _f32)],
    mesh=_mesh,
    compiler_params=_SCPARAMS,
    scratch_types=[
        pltpu.VMEM((4, 128), jnp.int32),
        pltpu.VMEM((512, 8), _f32),
        pltpu.VMEM_SHARED((NP, 8), _f32),
    ],
)
def _sc_scatter1(m1, dst2, x8, zn8, agg0, agg1, idx, mbuf, acc):
    c = lax.axis_index("c")
    s = lax.axis_index("s")
    w = s * 2 + c
    rows = pl.ds(s * 3200, 3200)

    @pl.when(c == 0)
    def _():
        pltpu.sync_copy(x8.at[rows, :], acc.at[rows, :])

    @pl.when(c == 1)
    def _():
        pltpu.sync_copy(zn8.at[rows, :], acc.at[rows, :])

    plsc.subcore_barrier()

    def body(i, _):
        rbase = w * RW + i * 4
        pltpu.sync_copy(dst2.at[pl.ds(rbase, 4), :], idx)
        pltpu.sync_copy(m1.at[pl.ds(rbase * 128, 512), :], mbuf)
        for j in range(4):
            pltpu.sync_copy(mbuf.at[pl.ds(j * 128, 128), :],
                            acc.at[idx.at[j]], add=True)
        return 0

    lax.fori_loop(0, RW // 4, body, 0)
    plsc.subcore_barrier()

    @pl.when(c == 0)
    def _():
        pltpu.sync_copy(acc.at[rows, :], agg0.at[rows, :])

    @pl.when(c == 1)
    def _():
        pltpu.sync_copy(acc.at[rows, :], agg1.at[rows, :])


_SKR = 8
_RPT = NR // 16
_SNCH = _RPT // _SKR


@functools.partial(
    pl.kernel,
    out_type=jax.ShapeDtypeStruct((NP, 64), _f32),
    mesh=_mesh,
    compiler_params=_SCPARAMS,
    scratch_types=[
        pltpu.VMEM((_SKR, 128), jnp.int32),
        pltpu.VMEM((_SKR, 128), jnp.int32),
        pltpu.VMEM((_SKR * 128, 16), _f32),
        pltpu.VMEM((_SKR * 128, 16), _f32),
        pltpu.VMEM_SHARED((NP, 16), _f32),
        pltpu.SemaphoreType.DMA,
        pltpu.SemaphoreType.DMA,
        pltpu.SemaphoreType.DMA,
    ],
)
def _sc_scatter23(m, dst2, h, accout,
                  ix0, ix1, mb0, mb1, acc, sem0, sem1, lsem):
    c = lax.axis_index("c")
    s = lax.axis_index("s")
    rows = pl.ds(s * 3200, 3200)

    for q in range(2):
        cols = pl.ds(c * 32 + q * 16, 16)

        pltpu.sync_copy(h.at[rows, cols], acc.at[rows, :])
        plsc.subcore_barrier()

        def load(chunk, ix, mb, cols=cols):
            rbase = s * _RPT + chunk * _SKR
            pltpu.sync_copy(dst2.at[pl.ds(rbase, _SKR), :], ix)
            return pltpu.async_copy(
                m.at[pl.ds(rbase * 128, _SKR * 128), cols], mb, lsem)

        def scat(ix, mb, sem):
            for j in range(_SKR):
                pltpu.async_copy(mb.at[pl.ds(j * 128, 128), :],
                                 acc.at[ix.at[j]], sem, add=True)

        def drain(ix, mb, sem):
            for j in range(_SKR):
                pltpu.make_async_copy(mb.at[pl.ds(j * 128, 128), :],
                                      acc.at[ix.at[j]], sem).wait()

        load(0, ix0, mb0).wait()

        def body(g, _):
            c0 = 2 * g
            ld = load(c0 + 1, ix1, mb1)
            scat(ix0, mb0, sem0)
            ld.wait()
            drain(ix0, mb0, sem0)

            @pl.when(c0 + 2 < _SNCH)
            def _():
                load(c0 + 2, ix0, mb0).wait()

            scat(ix1, mb1, sem1)
            drain(ix1, mb1, sem1)
            return 0

        lax.fori_loop(0, _SNCH // 2, body, 0)
        plsc.subcore_barrier()
        pltpu.sync_copy(acc.at[rows, :], accout.at[rows, cols])
        plsc.subcore_barrier()


_PREC = jax.lax.Precision.HIGHEST


def _dot(a, b, prec=_PREC):
    return jnp.dot(a, b, preferred_element_type=_f32, precision=prec)


def _sigmoid(x):
    return 1.0 / (1.0 + jnp.exp(-x))


def _softplus(x):
    return jnp.maximum(x, 0.0) + jnp.log1p(jnp.exp(-jnp.abs(x)))


def _nodeprep1_body(x8, w1d, w1s, d1, s1):
    xv = x8[...]
    d1[...] = _dot(xv, w1d[...])
    s1[...] = _dot(xv, w1s[...])


def _edge1_body(asum, ea, we1, b1, m1):
    a = asum[...] + _dot(ea[...], we1[...]) + b1[...]
    m1[...] = _sigmoid(a[:, 0:8]) * _softplus(a[:, 8:16])


def _proj_body(agg0, agg1, pw, pb, g, b, hout):
    xs = agg0[...] + agg1[...]
    h = _dot(xs, pw[...]) + pb[...]
    mu = jnp.mean(h, axis=1, keepdims=True)
    var = jnp.mean((h - mu) ** 2, axis=1, keepdims=True)
    h = (h - mu) / jnp.sqrt(var + 1e-5) * g[...] + b[...]
    hout[...] = jnp.maximum(h, 0.0)


def _dot3(a, b):
    ah = a.astype(jnp.bfloat16)
    al = (a - ah.astype(_f32)).astype(jnp.bfloat16)
    bh = b.astype(jnp.bfloat16)
    bl = (b - bh.astype(_f32)).astype(jnp.bfloat16)

    def d(x, y):
        return jax.lax.dot_general(x, y, (((1,), (0,)), ((), ())),
                                   preferred_element_type=_f32)

    return d(ah, bh) + d(ah, bl) + d(al, bh)


def _edge23_body(gg, ea, wcat, we, bb, mout):
    a = _dot3(gg[...], wcat[...]) + _dot3(ea[...], we[...]) + bb[...]
    m = _sigmoid(a[:, 0:64]) * _softplus(a[:, 64:128])
    mout[...] = jnp.concatenate([m, jnp.zeros_like(m)], axis=1)


def _act_body(a, hout):
    hout[...] = jnp.maximum(jnp.clip(a[...], -1e6, 1e6), 0.0)


def _poolact_body(a, bt, out):
    i = pl.program_id(0)
    h = jnp.maximum(jnp.clip(a[...], -1e6, 1e6), 0.0)
    haug = jnp.concatenate([h, jnp.ones((h.shape[0], 8), _f32)], axis=1)
    gids = bt[...].reshape(h.shape[0], 1)
    iota = jax.lax.broadcasted_iota(jnp.int32, (h.shape[0], NUM_GRAPHS), 1)
    onehot = (gids == iota).astype(_f32)
    contrib = jax.lax.dot_general(
        onehot, haug, (((0,), (0,)), ((), ())),
        preferred_element_type=_f32, precision=_PREC)

    @pl.when(i == 0)
    def _():
        out[...] = jnp.zeros_like(out)

    out[...] += contrib


def _head_body(pa, fw, fb, g, b, hw, hb, out):
    p = pa[...]
    pooled = p[:, 0:64] / jnp.maximum(p[:, 64:65], 1.0)
    gg = _dot(pooled, fw[...]) + fb[...]
    mu = jnp.mean(gg, axis=1, keepdims=True)
    var = jnp.mean((gg - mu) ** 2, axis=1, keepdims=True)
    gg = (gg - mu) / jnp.sqrt(var + 1e-5) * g[...] + b[...]
    gg = jnp.clip(jnp.maximum(gg, 0.0), -1e6, 1e6)
    out[...] = _dot(gg, hw[...]) + hb[...]


def _full(shape):
    return pl.BlockSpec(shape, lambda i: (0, 0))


def _nodeprep1(x8p, w1d, w1s):
    return pl.pallas_call(
        _nodeprep1_body,
        grid=(NG,),
        in_specs=[pl.BlockSpec((BN, 8), lambda i: (i, 0)),
                  _full((8, 16)), _full((8, 16))],
        out_specs=[pl.BlockSpec((BN, 16), lambda i: (i, 0))] * 2,
        out_shape=[jax.ShapeDtypeStruct((NP, 16), _f32)] * 2,
    )(x8p, w1d, w1s)


def _edge1(asum, eap, we1, b1):
    return pl.pallas_call(
        _edge1_body,
        grid=(EG,),
        in_specs=[pl.BlockSpec((BE, 16), lambda i: (i, 0)),
                  pl.BlockSpec((BE, 16), lambda i: (i, 0)),
                  _full((16, 16)), _full((1, 16))],
        out_specs=pl.BlockSpec((BE, 8), lambda i: (i, 0)),
        out_shape=jax.ShapeDtypeStruct((EP, 8), _f32),
    )(asum, eap, we1, b1)


def _proj(agg0, agg1, pw, pb, g, b):
    return pl.pallas_call(
        _proj_body,
        grid=(NG,),
        in_specs=[pl.BlockSpec((BN, 8), lambda i: (i, 0)),
                  pl.BlockSpec((BN, 8), lambda i: (i, 0)),
                  _full((8, 64)), _full((1, 64)), _full((1, 64)),
                  _full((1, 64))],
        out_specs=pl.BlockSpec((BN, 64), lambda i: (i, 0)),
        out_shape=jax.ShapeDtypeStruct((NP, 64), _f32),
    )(agg0, agg1, pw, pb, g, b)


def _edge23(gg, eap, wcat, we, bb):
    return pl.pallas_call(
        _edge23_body,
        grid=(EG,),
        in_specs=[pl.BlockSpec((BE, 128), lambda i: (i, 0)),
                  pl.BlockSpec((BE, 16), lambda i: (i, 0)),
                  _full((128, 128)), _full((16, 128)), _full((1, 128))],
        out_specs=pl.BlockSpec((BE, 128), lambda i: (i, 0)),
        out_shape=jax.ShapeDtypeStruct((EP, 128), _f32),
    )(gg, eap, wcat, we, bb)


def _act(a):
    nspec = pl.BlockSpec((BN, 64), lambda i: (i, 0))
    return pl.pallas_call(
        _act_body,
        grid=(NG,),
        in_specs=[nspec],
        out_specs=nspec,
        out_shape=jax.ShapeDtypeStruct((NP, 64), _f32),
    )(a)


def _poolact(a, bt):
    return pl.pallas_call(
        _poolact_body,
        grid=(25,),
        in_specs=[pl.BlockSpec((2000, 64), lambda i: (i, 0)),
                  pl.BlockSpec((1, 2000, 1), lambda i: (i, 0, 0))],
        out_specs=pl.BlockSpec((NUM_GRAPHS, 72), lambda i: (0, 0)),
        out_shape=jax.ShapeDtypeStruct((NUM_GRAPHS, 72), _f32),
    )(a, bt)


def _head(pa, fw, fb, g, b, hw, hb):
    return pl.pallas_call(
        _head_body,
        in_specs=[pl.BlockSpec((NUM_GRAPHS, 72), lambda: (0, 0)),
                  pl.BlockSpec((64, 64), lambda: (0, 0)),
                  pl.BlockSpec((1, 64), lambda: (0, 0)),
                  pl.BlockSpec((1, 64), lambda: (0, 0)),
                  pl.BlockSpec((1, 64), lambda: (0, 0)),
                  pl.BlockSpec((64, 8), lambda: (0, 0)),
                  pl.BlockSpec((1, 8), lambda: (0, 0))],
        out_specs=pl.BlockSpec((NUM_GRAPHS, 8), lambda: (0, 0)),
        out_shape=jax.ShapeDtypeStruct((NUM_GRAPHS, 8), _f32),
    )(pa, fw, fb, g, b, hw, hb)


def kernel(x, edge_index, edge_attr, batch, c1_Wf, c1_bf, c1_Ws, c1_bs,
           proj_W, proj_b, c2_Wf, c2_bf, c2_Ws, c2_bs, c3_Wf, c3_bf,
           c3_Ws, c3_bs, fc1_W, fc1_b, ln_g, ln_b, head_W, head_b):
    f32 = _f32
    x8p = jnp.zeros((NP, 8), f32).at[:N, :3].set(x)
    zn8 = jnp.zeros((NP, 8), f32)
    dst = edge_index[1].astype(jnp.int32)
    src = edge_index[0].astype(jnp.int32)
    padi = jnp.full((EP - E,), PAD_NODE, jnp.int32)
    dst2 = jnp.concatenate([dst, padi]).reshape(NR, 128)
    src2 = jnp.concatenate([src, padi]).reshape(NR, 128)
    eap = jnp.zeros((EP, 16), f32).at[:E, :].set(edge_attr)

    w1d = jnp.zeros((8, 16), f32).at[0:3, 0:3].set(c1_Wf[0:3]) \
                                 .at[0:3, 8:11].set(c1_Ws[0:3])
    w1s = jnp.zeros((8, 16), f32).at[0:3, 0:3].set(c1_Wf[3:6]) \
                                 .at[0:3, 8:11].set(c1_Ws[3:6])
    we1 = jnp.zeros((16, 16), f32).at[:, 0:3].set(c1_Wf[6:22]) \
                                  .at[:, 8:11].set(c1_Ws[6:22])
    b1 = jnp.zeros((1, 16), f32).at[0, 0:3].set(c1_bf).at[0, 8:11].set(c1_bs)

    pw8 = jnp.zeros((8, 64), f32).at[0:3, :].set(proj_W)
    pb = proj_b.reshape(1, 64)
    lg = ln_g.reshape(1, 64)
    lb = ln_b.reshape(1, 64)

    def wsplit(wf, wsm, bf, bs):
        wcat = jnp.concatenate(
            [jnp.concatenate([wf[0:64], wsm[0:64]], axis=1),
             jnp.concatenate([wf[64:128], wsm[64:128]], axis=1)], axis=0)
        we = jnp.concatenate([wf[128:144], wsm[128:144]], axis=1)
        bb = jnp.concatenate([bf, bs]).reshape(1, 128)
        return wcat, we, bb

    wcat2, we2, bb2 = wsplit(c2_Wf, c2_Ws, c2_bf, c2_bs)
    wcat3, we3, bb3 = wsplit(c3_Wf, c3_Ws, c3_bf, c3_bs)

    bt = batch.astype(jnp.int32).reshape(25, 2000, 1)
    fw = fc1_W
    fb = fc1_b.reshape(1, 64)
    hw8 = jnp.zeros((64, 8), f32).at[:, 0:5].set(head_W)
    hb8 = jnp.zeros((1, 8), f32).at[0, 0:5].set(head_b)

    d1, s1 = _nodeprep1(x8p, w1d, w1s)
    asum1 = _sc_gather1(d1, s1, dst2, src2)
    m1 = _edge1(asum1, eap, we1, b1)
    agg0, agg1 = _sc_scatter1(m1, dst2, x8p, zn8)
    h = _proj(agg0, agg1, pw8, pb, lg, lb)

    gg = _sc_gather23(h, dst2, src2)
    m2 = _edge23(gg, eap, wcat2, we2, bb2)
    a2 = _sc_scatter23(m2, dst2, h)
    h2 = _act(a2)

    gg = _sc_gather23(h2, dst2, src2)
    m3 = _edge23(gg, eap, wcat3, we3, bb3)
    a3 = _sc_scatter23(m3, dst2, h2)

    pa = _poolact(a3, bt)
    out8 = _head(pa, fw, fb, lg, lb, hw8, hb8)
    return out8[:, 0:5]
